# baseline (speedup 1.0000x reference)
import functools

import jax
import jax.numpy as jnp
from jax import lax
from jax.experimental import pallas as pl
from jax.experimental.pallas import tpu as pltpu
from jax.experimental.pallas import tpu_sc as plsc

N = 10000
E = 160000
D = 256
F1 = 96
F2 = 48
FP = 128
ZDIM = 128

NC = 2
NS = 16
NW = NC * NS
LANES = 16
CHUNK = 64
NBUF = 4
NIB = 8
FAST_CORE = 0
NCH_F = 156
NCH_S = 4
TOT_CHUNKS = NS * (NCH_F + NCH_S)
E_PAD = TOT_CHUNKS * CHUNK
NP = 10240
ROWS_PER_SUB = NP // NS


def _make_spmm(F):
    nfeat = F // LANES
    mesh = plsc.VectorSubcoreMesh(core_axis_name="c", subcore_axis_name="s")

    @functools.partial(
        pl.kernel,
        out_type=jax.ShapeDtypeStruct((NC, NP, F), jnp.float32),
        mesh=mesh,
        scratch_types=[
            [pltpu.VMEM((3, CHUNK), jnp.int32)] * NIB,
            [pltpu.VMEM((CHUNK, F), jnp.float32)] * NBUF,
            [pltpu.SemaphoreType.DMA] * NIB,
            [pltpu.SemaphoreType.DMA] * NBUF,
            [pltpu.SemaphoreType.DMA] * NBUF,
            pltpu.VMEM_SHARED((NP, F), jnp.float32),
        ],
    )
    def spmm(m_hbm, ech_hbm, zeros_hbm, out_hbm,
             idxr, buf, isem, gsem, ssem, acc):
        c = lax.axis_index("c")
        s = lax.axis_index("s")
        row0 = s * ROWS_PER_SUB
        on_fast = c == FAST_CORE
        nch = jnp.where(on_fast, NCH_F, NCH_S)
        nocts = jnp.where(on_fast, NCH_F // NIB, NCH_S // NIB)
        base = jnp.where(on_fast, s * NCH_F, NS * NCH_F + s * NCH_S)

        pltpu.sync_copy(zeros_hbm, acc.at[pl.ds(row0, ROWS_PER_SUB)])
        plsc.subcore_barrier()

        for k in range(4):
            pltpu.async_copy(ech_hbm.at[base + k], idxr[k], isem[k])
        for b in range(2):
            pltpu.make_async_copy(ech_hbm.at[base + b], idxr[b],
                                  isem[b]).wait()
            pltpu.async_copy(m_hbm.at[idxr[b].at[0]], buf[b], gsem[b])

        def oct_body(jj, carry):
            for q in range(NIB):
                j = NIB * jj + q
                b = q % NBUF
                ib = q
                pltpu.make_async_copy(m_hbm.at[idxr[ib].at[0]],
                                      buf[b], gsem[b]).wait()

                def group_body(g, carry2):
                    wv16 = lax.bitcast_convert_type(
                        idxr[ib][2, pl.ds(g * LANES, LANES)], jnp.float32)
                    for l in range(LANES):
                        wvec = jnp.full((LANES,), wv16[l], jnp.float32)
                        e = g * LANES + l
                        for t in range(nfeat):
                            sl = pl.ds(t * LANES, LANES)
                            buf[b][e, sl] = buf[b][e, sl] * wvec
                    return carry2

                lax.fori_loop(0, CHUNK // LANES, group_body, 0)
                pltpu.async_copy(buf[b], acc.at[idxr[ib].at[1]], ssem[b],
                                 add=True)

                jn = j + 2
                bp = (q + 2) % NBUF
                ibn = (q + 2) % NIB

                @pl.when(jn < nch)
                def _():
                    @pl.when(j >= 2)
                    def _():
                        pltpu.make_async_copy(
                            buf[bp], acc.at[idxr[ibn].at[1]],
                            ssem[bp]).wait()
                    pltpu.make_async_copy(ech_hbm.at[base + jn], idxr[ibn],
                                          isem[ibn]).wait()
                    pltpu.async_copy(m_hbm.at[idxr[ibn].at[0]], buf[bp],
                                     gsem[bp])

                jm = j + 4
                ibm = (q + 4) % NIB

                @pl.when(jm < nch)
                def _():
                    pltpu.async_copy(ech_hbm.at[base + jm], idxr[ibm],
                                     isem[ibm])
            return carry

        lax.fori_loop(0, nocts, oct_body, 0)

        for q in range(4):
            j = nch - 4 + q
            b = q
            ib = q
            pltpu.make_async_copy(m_hbm.at[idxr[ib].at[0]],
                                  buf[b], gsem[b]).wait()

            def tail_group(g, carry2, ib=ib, b=b):
                wv16 = lax.bitcast_convert_type(
                    idxr[ib][2, pl.ds(g * LANES, LANES)], jnp.float32)
                for l in range(LANES):
                    wvec = jnp.full((LANES,), wv16[l], jnp.float32)
                    e = g * LANES + l
                    for t in range(nfeat):
                        sl = pl.ds(t * LANES, LANES)
                        buf[b][e, sl] = buf[b][e, sl] * wvec
                return carry2

            lax.fori_loop(0, CHUNK // LANES, tail_group, 0)
            pltpu.async_copy(buf[b], acc.at[idxr[ib].at[1]], ssem[b],
                             add=True)
            jn = j + 2
            bp = (q + 2) % NBUF
            ibn = (q + 2) % NIB

            @pl.when(jn < nch)
            def _():
                @pl.when(j >= 2)
                def _():
                    pltpu.make_async_copy(
                        buf[bp], acc.at[idxr[ibn].at[1]], ssem[bp]).wait()
                pltpu.make_async_copy(ech_hbm.at[base + jn], idxr[ibn],
                                      isem[ibn]).wait()
                pltpu.async_copy(m_hbm.at[idxr[ibn].at[0]], buf[bp],
                                 gsem[bp])

        for q in range(4):
            pltpu.make_async_copy(
                buf[q], acc.at[idxr[q].at[1]], ssem[q]).wait()
        plsc.subcore_barrier()

        pltpu.sync_copy(acc.at[pl.ds(row0, ROWS_PER_SUB)],
                        out_hbm.at[c, pl.ds(row0, ROWS_PER_SUB)])

    return spmm


_spmm = _make_spmm(FP)


def _mm_body(x_ref, w_ref, o_ref):
    o_ref[...] = jnp.dot(x_ref[...], w_ref[...],
                         preferred_element_type=jnp.float32)


def _dense_mm(x, w, bm):
    m, k = x.shape
    n = w.shape[1]
    return pl.pallas_call(
        _mm_body,
        grid=(pl.cdiv(m, bm),),
        in_specs=[pl.BlockSpec((bm, k), lambda i: (i, 0)),
                  pl.BlockSpec((k, n), lambda i: (0, 0))],
        out_specs=pl.BlockSpec((bm, n), lambda i: (i, 0)),
        out_shape=jax.ShapeDtypeStruct((m, n), jnp.float32),
    )(x, w)


def _part_mm_body(relu, p_ref, w_ref, o_ref):
    h = p_ref[0] + p_ref[1]
    if relu:
        h = jnp.maximum(h, 0.0)
    o_ref[...] = jnp.dot(h, w_ref[...], preferred_element_type=jnp.float32)


def _partial_mm(p, w, bm, relu):
    _, m, k = p.shape
    n = w.shape[1]
    return pl.pallas_call(
        functools.partial(_part_mm_body, relu),
        grid=(pl.cdiv(m, bm),),
        in_specs=[pl.BlockSpec((2, bm, k), lambda i: (0, i, 0)),
                  pl.BlockSpec((k, n), lambda i: (0, 0))],
        out_specs=pl.BlockSpec((bm, n), lambda i: (i, 0)),
        out_shape=jax.ShapeDtypeStruct((m, n), jnp.float32),
    )(p, w)


def _gram_body(a_ref, b_ref, o_ref):
    o_ref[...] = lax.dot_general(
        a_ref[...], b_ref[...], (((1,), (1,)), ((), ())),
        preferred_element_type=jnp.float32)


def _gram_bf16(z, bm, bn):
    m, k = z.shape
    zb = z.astype(jnp.bfloat16)
    return pl.pallas_call(
        _gram_body,
        grid=(pl.cdiv(m, bm), pl.cdiv(m, bn)),
        in_specs=[pl.BlockSpec((bm, k), lambda i, j: (i, 0)),
                  pl.BlockSpec((bn, k), lambda i, j: (j, 0))],
        out_specs=pl.BlockSpec((bm, bn), lambda i, j: (i, j)),
        out_shape=jax.ShapeDtypeStruct((m, m), jnp.float32),
    )(zb, zb)


def _gram(z, bm, bn):
    m, k = z.shape
    return pl.pallas_call(
        _gram_body,
        grid=(pl.cdiv(m, bm), pl.cdiv(m, bn)),
        in_specs=[pl.BlockSpec((bm, k), lambda i, j: (i, 0)),
                  pl.BlockSpec((bn, k), lambda i, j: (j, 0))],
        out_specs=pl.BlockSpec((bm, bn), lambda i, j: (i, j)),
        out_shape=jax.ShapeDtypeStruct((m, m), jnp.float32),
    )(z, z)


def kernel(features, edge_index, edge_weight,
           W11, W21, W31, W12, W22, W32, Wf1, Wf2, Wf3):
    w_cat = jnp.concatenate([W11, W21, W31], axis=1)
    w_cat = jnp.pad(w_cat, ((0, 0), (0, FP - F1)))
    h1, h2 = W12.shape
    zero = jnp.zeros((h1, h2), jnp.float32)
    w_bd = jnp.concatenate([
        jnp.concatenate([W12, zero, zero], axis=1),
        jnp.concatenate([zero, W22, zero], axis=1),
        jnp.concatenate([zero, zero, W32], axis=1),
    ], axis=0)
    w_bd = jnp.pad(w_bd, ((0, FP - F1), (0, FP - F2)))
    w_f = jnp.concatenate([Wf1, Wf2, Wf3], axis=0) / 3.0
    w_f = jnp.pad(w_f, ((0, FP - F2), (0, 0)))

    pad = E_PAD - E
    src = jnp.concatenate([edge_index[0], jnp.zeros((pad,), jnp.int32)])
    dst = jnp.concatenate([edge_index[1], jnp.zeros((pad,), jnp.int32)])
    ew = jnp.concatenate([edge_weight, jnp.zeros((pad,), jnp.float32)])
    ech = jnp.stack([src.reshape(TOT_CHUNKS, CHUNK),
                     dst.reshape(TOT_CHUNKS, CHUNK),
                     lax.bitcast_convert_type(ew, jnp.int32)
                        .reshape(TOT_CHUNKS, CHUNK)], axis=1)
    zrows = jnp.zeros((ROWS_PER_SUB, FP), jnp.float32)

    m1 = _dense_mm(features, w_cat, 512)
    p1 = _spmm(m1, ech, zrows)
    m2 = _partial_mm(p1, w_bd, 512, relu=True)
    p2 = _spmm(m2, ech, zrows)
    zm = _partial_mm(p2, w_f, 512, relu=False)[:N]
    recon = _gram(zm, 2048, 2048)
    return recon.reshape(-1)


# speedup vs baseline: 1.0110x; 1.0110x over previous
import functools

import jax
import jax.numpy as jnp
from jax import lax
from jax.experimental import pallas as pl
from jax.experimental.pallas import tpu as pltpu
from jax.experimental.pallas import tpu_sc as plsc

N = 10000
E = 160000
D = 256
F1 = 96
F2 = 48
FP = 128
ZDIM = 128

NC = 2
NS = 16
NW = NC * NS
LANES = 16
CHUNK = 64
NBUF = 4
NIB = 8
FAST_CORE = 0
NCH_F = 156
NCH_S = 4
TOT_CHUNKS = NS * (NCH_F + NCH_S)
E_PAD = TOT_CHUNKS * CHUNK
NP = 10240
ROWS_PER_SUB = NP // NS


def _make_spmm(F):
    nfeat = F // LANES
    mesh = plsc.VectorSubcoreMesh(core_axis_name="c", subcore_axis_name="s")

    @functools.partial(
        pl.kernel,
        out_type=jax.ShapeDtypeStruct((NC, NP, F), jnp.float32),
        mesh=mesh,
        scratch_types=[
            [pltpu.VMEM((3, CHUNK), jnp.int32)] * NIB,
            [pltpu.VMEM((CHUNK, F), jnp.float32)] * NBUF,
            [pltpu.SemaphoreType.DMA] * NIB,
            [pltpu.SemaphoreType.DMA] * NBUF,
            [pltpu.SemaphoreType.DMA] * NBUF,
            pltpu.VMEM_SHARED((NP, F), jnp.float32),
        ],
    )
    def spmm(m_hbm, ech_hbm, out_hbm,
             idxr, buf, isem, gsem, ssem, acc):
        c = lax.axis_index("c")
        s = lax.axis_index("s")
        row0 = s * ROWS_PER_SUB
        on_fast = c == FAST_CORE
        nch = jnp.where(on_fast, NCH_F, NCH_S)
        nocts = jnp.where(on_fast, NCH_F // NIB, NCH_S // NIB)
        base = jnp.where(on_fast, s * NCH_F, NS * NCH_F + s * NCH_S)

        def zrow_body(r, carry):
            for t in range(nfeat):
                buf[0][r, pl.ds(t * LANES, LANES)] = jnp.zeros(
                    (LANES,), jnp.float32)
            return carry

        lax.fori_loop(0, CHUNK, zrow_body, 0)

        def zcp_body(k, carry):
            pltpu.sync_copy(buf[0],
                            acc.at[pl.ds(row0 + k * CHUNK, CHUNK)])
            return carry

        lax.fori_loop(0, ROWS_PER_SUB // CHUNK, zcp_body, 0)
        plsc.subcore_barrier()

        for k in range(4):
            pltpu.async_copy(ech_hbm.at[base + k], idxr[k], isem[k])
        for b in range(2):
            pltpu.make_async_copy(ech_hbm.at[base + b], idxr[b],
                                  isem[b]).wait()
            pltpu.async_copy(m_hbm.at[idxr[b].at[0]], buf[b], gsem[b])

        def oct_body(jj, carry):
            for q in range(NIB):
                j = NIB * jj + q
                b = q % NBUF
                ib = q
                pltpu.make_async_copy(m_hbm.at[idxr[ib].at[0]],
                                      buf[b], gsem[b]).wait()

                def group_body(g, carry2, ib=ib, b=b):
                    wv16 = lax.bitcast_convert_type(
                        idxr[ib][2, pl.ds(g * LANES, LANES)], jnp.float32)
                    for l in range(LANES):
                        wvec = jnp.full((LANES,), wv16[l], jnp.float32)
                        e = g * LANES + l
                        for t in range(nfeat):
                            sl = pl.ds(t * LANES, LANES)
                            buf[b][e, sl] = buf[b][e, sl] * wvec
                    return carry2

                lax.fori_loop(0, CHUNK // LANES, group_body, 0)
                pltpu.async_copy(buf[b], acc.at[idxr[ib].at[1]], ssem[b],
                                 add=True)

                jn = j + 2
                bp = (q + 2) % NBUF
                ibn = (q + 2) % NIB

                @pl.when(jn < nch)
                def _():
                    @pl.when(j >= 2)
                    def _():
                        pltpu.make_async_copy(
                            buf[bp], acc.at[idxr[ibn].at[1]],
                            ssem[bp]).wait()
                    pltpu.make_async_copy(ech_hbm.at[base + jn], idxr[ibn],
                                          isem[ibn]).wait()
                    pltpu.async_copy(m_hbm.at[idxr[ibn].at[0]], buf[bp],
                                     gsem[bp])

                jm = j + 4
                ibm = (q + 4) % NIB

                @pl.when(jm < nch)
                def _():
                    pltpu.async_copy(ech_hbm.at[base + jm], idxr[ibm],
                                     isem[ibm])
            return carry

        lax.fori_loop(0, nocts, oct_body, 0)

        for q in range(4):
            j = nch - 4 + q
            b = q
            ib = q
            pltpu.make_async_copy(m_hbm.at[idxr[ib].at[0]],
                                  buf[b], gsem[b]).wait()

            def tail_group(g, carry2, ib=ib, b=b):
                wv16 = lax.bitcast_convert_type(
                    idxr[ib][2, pl.ds(g * LANES, LANES)], jnp.float32)
                for l in range(LANES):
                    wvec = jnp.full((LANES,), wv16[l], jnp.float32)
                    e = g * LANES + l
                    for t in range(nfeat):
                        sl = pl.ds(t * LANES, LANES)
                        buf[b][e, sl] = buf[b][e, sl] * wvec
                return carry2

            lax.fori_loop(0, CHUNK // LANES, tail_group, 0)
            pltpu.async_copy(buf[b], acc.at[idxr[ib].at[1]], ssem[b],
                             add=True)
            jn = j + 2
            bp = (q + 2) % NBUF
            ibn = (q + 2) % NIB

            @pl.when(jn < nch)
            def _():
                @pl.when(j >= 2)
                def _():
                    pltpu.make_async_copy(
                        buf[bp], acc.at[idxr[ibn].at[1]], ssem[bp]).wait()
                pltpu.make_async_copy(ech_hbm.at[base + jn], idxr[ibn],
                                      isem[ibn]).wait()
                pltpu.async_copy(m_hbm.at[idxr[ibn].at[0]], buf[bp],
                                 gsem[bp])

        for q in range(4):
            pltpu.make_async_copy(
                buf[q], acc.at[idxr[q].at[1]], ssem[q]).wait()
        plsc.subcore_barrier()

        pltpu.sync_copy(acc.at[pl.ds(row0, ROWS_PER_SUB)],
                        out_hbm.at[c, pl.ds(row0, ROWS_PER_SUB)])

    return spmm


_spmm = _make_spmm(FP)


def _mm_body(x_ref, w_ref, o_ref):
    o_ref[...] = jnp.dot(x_ref[...], w_ref[...],
                         preferred_element_type=jnp.float32)


def _dense_mm(x, w, bm):
    m, k = x.shape
    n = w.shape[1]
    return pl.pallas_call(
        _mm_body,
        grid=(pl.cdiv(m, bm),),
        in_specs=[pl.BlockSpec((bm, k), lambda i: (i, 0)),
                  pl.BlockSpec((k, n), lambda i: (0, 0))],
        out_specs=pl.BlockSpec((bm, n), lambda i: (i, 0)),
        out_shape=jax.ShapeDtypeStruct((m, n), jnp.float32),
    )(x, w)


def _part_mm_body(relu, p_ref, w_ref, o_ref):
    h = p_ref[0] + p_ref[1]
    if relu:
        h = jnp.maximum(h, 0.0)
    o_ref[...] = jnp.dot(h, w_ref[...], preferred_element_type=jnp.float32)


def _partial_mm(p, w, bm, relu):
    _, m, k = p.shape
    n = w.shape[1]
    return pl.pallas_call(
        functools.partial(_part_mm_body, relu),
        grid=(pl.cdiv(m, bm),),
        in_specs=[pl.BlockSpec((2, bm, k), lambda i: (0, i, 0)),
                  pl.BlockSpec((k, n), lambda i: (0, 0))],
        out_specs=pl.BlockSpec((bm, n), lambda i: (i, 0)),
        out_shape=jax.ShapeDtypeStruct((m, n), jnp.float32),
    )(p, w)


def _gram_body(a_ref, b_ref, o_ref):
    o_ref[...] = lax.dot_general(
        a_ref[...], b_ref[...], (((1,), (1,)), ((), ())),
        preferred_element_type=jnp.float32)


def _gram_bf16(z, bm, bn):
    m, k = z.shape
    zb = z.astype(jnp.bfloat16)
    return pl.pallas_call(
        _gram_body,
        grid=(pl.cdiv(m, bm), pl.cdiv(m, bn)),
        in_specs=[pl.BlockSpec((bm, k), lambda i, j: (i, 0)),
                  pl.BlockSpec((bn, k), lambda i, j: (j, 0))],
        out_specs=pl.BlockSpec((bm, bn), lambda i, j: (i, j)),
        out_shape=jax.ShapeDtypeStruct((m, m), jnp.float32),
    )(zb, zb)


def _gram(z, bm, bn):
    m, k = z.shape
    return pl.pallas_call(
        _gram_body,
        grid=(pl.cdiv(m, bm), pl.cdiv(m, bn)),
        in_specs=[pl.BlockSpec((bm, k), lambda i, j: (i, 0)),
                  pl.BlockSpec((bn, k), lambda i, j: (j, 0))],
        out_specs=pl.BlockSpec((bm, bn), lambda i, j: (i, j)),
        out_shape=jax.ShapeDtypeStruct((m, m), jnp.float32),
    )(z, z)


def kernel(features, edge_index, edge_weight,
           W11, W21, W31, W12, W22, W32, Wf1, Wf2, Wf3):
    w_cat = jnp.concatenate([W11, W21, W31], axis=1)
    w_cat = jnp.pad(w_cat, ((0, 0), (0, FP - F1)))
    h1, h2 = W12.shape
    zero = jnp.zeros((h1, h2), jnp.float32)
    w_bd = jnp.concatenate([
        jnp.concatenate([W12, zero, zero], axis=1),
        jnp.concatenate([zero, W22, zero], axis=1),
        jnp.concatenate([zero, zero, W32], axis=1),
    ], axis=0)
    w_bd = jnp.pad(w_bd, ((0, FP - F1), (0, FP - F2)))
    w_f = jnp.concatenate([Wf1, Wf2, Wf3], axis=0) / 3.0
    w_f = jnp.pad(w_f, ((0, FP - F2), (0, 0)))

    pad = E_PAD - E
    src = jnp.concatenate([edge_index[0], jnp.zeros((pad,), jnp.int32)])
    dst = jnp.concatenate([edge_index[1], jnp.zeros((pad,), jnp.int32)])
    ew = jnp.concatenate([edge_weight, jnp.zeros((pad,), jnp.float32)])
    ech = jnp.stack([src.reshape(TOT_CHUNKS, CHUNK),
                     dst.reshape(TOT_CHUNKS, CHUNK),
                     lax.bitcast_convert_type(ew, jnp.int32)
                        .reshape(TOT_CHUNKS, CHUNK)], axis=1)

    m1 = _dense_mm(features, w_cat, 512)
    p1 = _spmm(m1, ech)
    m2 = _partial_mm(p1, w_bd, 512, relu=True)
    p2 = _spmm(m2, ech)
    zm = _partial_mm(p2, w_f, 512, relu=False)[:N]
    recon = _gram(zm, 2048, 2048)
    return recon.reshape(-1)
